# baseline (speedup 1.0000x reference)
import jax
import jax.numpy as jnp
from jax.experimental import pallas as pl
from jax.experimental.pallas import tpu as pltpu

_A = 20
_ATD = 6
_OC = _A * _ATD
_DIM = 512
_IN = 1024
_B, _H, _W = 8, 37, 37
_HW = _H * _W
_PW = _W + 3
_NP = 1536
_XL = (_H + 4) * _PW


def _body(x_ref, w1_ref, b1_ref, w2_ref, b2_ref, o_ref, xp_ref):
    b = pl.program_id(0)
    parity = jax.lax.rem(b, 2)

    @pl.when(b == 0)
    def _():
        xp_ref[...] = jnp.zeros((2, _IN, _XL), jnp.bfloat16)

    @pl.when(b < _B)
    def _():
        for h in range(_H):
            xp_ref[parity, :, h * _PW + _PW + 1:h * _PW + _PW + 1 + _W] = (
                x_ref[0, :, h * _W:(h + 1) * _W])

    @pl.when(b > 0)
    def _():
        prev = 1 - parity
        acc = jnp.zeros((_DIM, _NP), jnp.float32)
        for t in range(9):
            off = (t // 3) * _PW + (t % 3)
            acc = acc + jnp.dot(
                w1_ref[t], xp_ref[prev, :, off:off + _NP],
                preferred_element_type=jnp.float32)
        acc = acc + b1_ref[...]
        y = jnp.clip(acc, 0.0, 6.0).astype(jnp.bfloat16)
        z = jax.lax.dot_general(
            y, w2_ref[...], (((0,), (0,)), ((), ())),
            preferred_element_type=jnp.float32)
        z = z + b2_ref[...]
        for h in range(_H):
            o_ref[0, h * _W:(h + 1) * _W, :] = z[h * _PW:h * _PW + _W, :]


def kernel(fmap, W1, b1, W2, b2):
    xc = fmap.reshape(_B, _IN, _HW).astype(jnp.bfloat16)
    w1 = jnp.transpose(W1, (2, 3, 0, 1)).reshape(9, _DIM, _IN)
    w1 = w1.astype(jnp.bfloat16)
    w2 = W2.reshape(_OC, _DIM).T.astype(jnp.bfloat16)
    b1c = b1.reshape(_DIM, 1)
    b2c = b2.reshape(1, _OC)

    out = pl.pallas_call(
        _body,
        grid=(_B + 1,),
        in_specs=[
            pl.BlockSpec((1, _IN, _HW),
                         lambda b: (jnp.minimum(b, _B - 1), 0, 0)),
            pl.BlockSpec((9, _DIM, _IN), lambda b: (0, 0, 0)),
            pl.BlockSpec((_DIM, 1), lambda b: (0, 0)),
            pl.BlockSpec((_DIM, _OC), lambda b: (0, 0)),
            pl.BlockSpec((1, _OC), lambda b: (0, 0)),
        ],
        out_specs=pl.BlockSpec(
            (1, _HW, _OC), lambda b: (jnp.maximum(b - 1, 0), 0, 0)),
        out_shape=jax.ShapeDtypeStruct((_B, _HW, _OC), jnp.float32),
        scratch_shapes=[pltpu.VMEM((2, _IN, _XL), jnp.bfloat16)],
    )(xc, w1, b1c, w2, b2c)

    return out.reshape(_B, _H, _W, _A, _ATD)


# speedup vs baseline: 1.1002x; 1.1002x over previous
import jax
import jax.numpy as jnp
from jax.experimental import pallas as pl
from jax.experimental.pallas import tpu as pltpu

_A = 20
_ATD = 6
_OC = _A * _ATD
_DIM = 512
_IN = 1024
_B, _H, _W = 8, 37, 37
_HW = _H * _W
_PW = _W + 3
_NP = 1536
_XL = (_H + 4) * _PW


def _body(x_ref, w1_ref, b1_ref, w2_ref, b2_ref, o_ref, w1s_ref):
    @pl.when(pl.program_id(0) == 0)
    def _():
        w1s_ref[...] = w1_ref[...].astype(jnp.bfloat16).reshape(
            9, _DIM, _IN)

    acc = jnp.zeros((_DIM, _NP), jnp.float32)
    for t in range(9):
        off = (t // 3) * _PW + (t % 3)
        acc = acc + jnp.dot(
            w1s_ref[t], x_ref[0, :, off:off + _NP],
            preferred_element_type=jnp.float32)
    acc = acc + b1_ref[...]
    y = jnp.clip(acc, 0.0, 6.0).astype(jnp.bfloat16)
    z = jax.lax.dot_general(
        y, w2_ref[...], (((0,), (0,)), ((), ())),
        preferred_element_type=jnp.float32)
    z = z + b2_ref[...]
    for h in range(_H):
        o_ref[0, h * _W:(h + 1) * _W, :] = z[h * _PW:h * _PW + _W, :]


def kernel(fmap, W1, b1, W2, b2):
    xp = jnp.pad(fmap, ((0, 0), (0, 0), (1, 3), (1, 2)))
    xf = xp.reshape(_B, _IN, _XL).astype(jnp.bfloat16)

    w1 = jnp.transpose(W1, (2, 3, 0, 1))
    w2 = W2.reshape(_OC, _DIM).T.astype(jnp.bfloat16)
    b1c = b1.reshape(_DIM, 1)
    b2c = b2.reshape(1, _OC)

    out = pl.pallas_call(
        _body,
        grid=(_B,),
        in_specs=[
            pl.BlockSpec((1, _IN, _XL), lambda b: (b, 0, 0)),
            pl.BlockSpec((3, 3, _DIM, _IN), lambda b: (0, 0, 0, 0)),
            pl.BlockSpec((_DIM, 1), lambda b: (0, 0)),
            pl.BlockSpec((_DIM, _OC), lambda b: (0, 0)),
            pl.BlockSpec((1, _OC), lambda b: (0, 0)),
        ],
        out_specs=pl.BlockSpec((1, _HW, _OC), lambda b: (b, 0, 0)),
        out_shape=jax.ShapeDtypeStruct((_B, _HW, _OC), jnp.float32),
        scratch_shapes=[pltpu.VMEM((9, _DIM, _IN), jnp.bfloat16)],
    )(xf, w1, b1c, w2, b2c)

    return out.reshape(_B, _H, _W, _A, _ATD)


# speedup vs baseline: 1.1039x; 1.0033x over previous
import jax
import jax.numpy as jnp
from jax.experimental import pallas as pl
from jax.experimental.pallas import tpu as pltpu

_A = 20
_ATD = 6
_OC = _A * _ATD
_DIM = 512
_IN = 1024
_B, _H, _W = 8, 37, 37
_HW = _H * _W
_PW = _W + 3
_NP = 1536
_XL = (_H + 4) * _PW


def _body(x_ref, w1_ref, b1_ref, w2_ref, b2_ref, o_ref, w1s_ref):
    @pl.when(pl.program_id(0) == 0)
    def _():
        w1s_ref[...] = w1_ref[...].astype(jnp.bfloat16).reshape(
            9, _DIM, _IN)

    acc0 = jnp.zeros((_DIM, _NP), jnp.float32)
    acc1 = jnp.zeros((_DIM, _NP), jnp.float32)
    for t in range(9):
        off = (t // 3) * _PW + (t % 3)
        d = jnp.dot(
            w1s_ref[t], x_ref[0, :, off:off + _NP],
            preferred_element_type=jnp.float32)
        if t % 2 == 0:
            acc0 = acc0 + d
        else:
            acc1 = acc1 + d
    acc = acc0 + acc1 + b1_ref[...]
    y = jnp.clip(acc, 0.0, 6.0).astype(jnp.bfloat16)
    z = jax.lax.dot_general(
        y, w2_ref[...], (((0,), (0,)), ((), ())),
        preferred_element_type=jnp.float32)
    z = z + b2_ref[...]
    for h in range(_H):
        o_ref[0, h * _W:(h + 1) * _W, :] = z[h * _PW:h * _PW + _W, :]


def kernel(fmap, W1, b1, W2, b2):
    xp = jnp.pad(fmap, ((0, 0), (0, 0), (1, 3), (1, 2)))
    xf = xp.reshape(_B, _IN, _XL).astype(jnp.bfloat16)

    w1 = jnp.transpose(W1, (2, 3, 0, 1))
    w2 = W2.reshape(_OC, _DIM).T.astype(jnp.bfloat16)
    b1c = b1.reshape(_DIM, 1)
    b2c = b2.reshape(1, _OC)

    out = pl.pallas_call(
        _body,
        grid=(_B,),
        in_specs=[
            pl.BlockSpec((1, _IN, _XL), lambda b: (b, 0, 0)),
            pl.BlockSpec((3, 3, _DIM, _IN), lambda b: (0, 0, 0, 0)),
            pl.BlockSpec((_DIM, 1), lambda b: (0, 0)),
            pl.BlockSpec((_DIM, _OC), lambda b: (0, 0)),
            pl.BlockSpec((1, _OC), lambda b: (0, 0)),
        ],
        out_specs=pl.BlockSpec((1, _HW, _OC), lambda b: (b, 0, 0)),
        out_shape=jax.ShapeDtypeStruct((_B, _HW, _OC), jnp.float32),
        scratch_shapes=[pltpu.VMEM((9, _DIM, _IN), jnp.bfloat16)],
    )(xf, w1, b1c, w2, b2c)

    return out.reshape(_B, _H, _W, _A, _ATD)
